# baseline (speedup 1.0000x reference)
import functools

import jax
import jax.numpy as jnp
from jax import lax
from jax.experimental import pallas as pl
from jax.experimental.pallas import tpu as pltpu
from jax.experimental.pallas import tpu_sc as plsc

GRID_N = 16
NV = GRID_N * GRID_N
D = 768
B = 65536
NC, NS = 2, 16
NW = NC * NS
PER_W = B // NW
CH3 = 32
NCH3 = PER_W // CH3
LAG = 4

_MESH = plsc.VectorSubcoreMesh(core_axis_name="c", subcore_axis_name="s")


@functools.partial(
    pl.kernel,
    mesh=_MESH,
    out_type=jax.ShapeDtypeStruct((B, D), jnp.float32),
    scratch_types=[
        pltpu.VMEM_SHARED((NV * D,), jnp.float32),
        pltpu.VMEM((PER_W,), jnp.int32),
        pltpu.VMEM((2, GRID_N, D), jnp.float32),
        pltpu.VMEM((GRID_N,), jnp.int32),
        pltpu.VMEM((GRID_N,), jnp.int32),
        pltpu.SemaphoreType.DMA,
        pltpu.SemaphoreType.DMA,
        pltpu.SemaphoreType.DMA,
    ],
)
def _embed(pos_hbm, crd0_hbm, crd1_hbm, emb0_hbm, emb1_hbm, out_hbm,
           table_sh, idx_v, bufs, crd0_v, crd1_v, sf0, sf1, swr):
    cid = lax.axis_index("c")
    sid = lax.axis_index("s")
    wid = sid * NC + cid
    row0 = wid * PER_W

    t0 = sid * GRID_N
    pltpu.sync_copy(crd0_hbm.at[pl.ds(t0, GRID_N)], crd0_v)
    pltpu.sync_copy(crd1_hbm.at[pl.ds(t0, GRID_N)], crd1_v)
    c0 = crd0_v[...]
    c1 = crd1_v[...]
    cp0 = pltpu.async_copy(emb0_hbm.at[c0], bufs.at[0], sf0)
    cp1 = pltpu.async_copy(emb1_hbm.at[c1], bufs.at[1], sf1)
    pltpu.sync_copy(pos_hbm.at[pl.ds(row0, PER_W)], idx_v)
    cp0.wait()
    cp1.wait()

    def addrow(r, carry):
        for f in range(D // 16):
            sl = pl.ds(f * 16, 16)
            bufs[0, r, sl] = bufs[0, r, sl] + bufs[1, r, sl]
        return carry

    lax.fori_loop(0, GRID_N, addrow, 0)

    def stage(r, carry):
        pltpu.sync_copy(bufs.at[0, r], table_sh.at[pl.ds((t0 + r) * D, D)])
        return carry

    lax.fori_loop(0, GRID_N, stage, 0)
    plsc.subcore_barrier()

    def issue(g, carry):
        cbase = g * CH3
        def qbody(q, carry2):
            pvec = idx_v[pl.ds(cbase + q * 16, 16)]
            for j in range(16):
                p = pvec[j] * D
                pltpu.async_copy(table_sh.at[pl.ds(p, D)],
                                 out_hbm.at[row0 + cbase + q * 16 + j], swr)
            return carry2
        lax.fori_loop(0, CH3 // 16, qbody, 0)
        return carry

    def drain1(carry):
        pltpu.make_async_copy(pos_hbm.at[pl.ds(0, CH3 * D)],
                              table_sh.at[pl.ds(0, CH3 * D)], swr).wait()
        return carry

    def body(g, carry):
        issue(g, carry)

        @pl.when(g >= LAG)
        def _():
            drain1(0)
        return carry

    lax.fori_loop(0, NCH3, body, 0)
    lax.fori_loop(0, LAG, lambda k, c: drain1(c), 0)


def kernel(positions, coords, emb0, emb1):
    pos = positions.astype(jnp.int32)
    crd = coords.astype(jnp.int32)
    return _embed(pos, crd[:, 0].reshape(-1), crd[:, 1].reshape(-1),
                  emb0.astype(jnp.float32), emb1.astype(jnp.float32))


# speedup vs baseline: 1.5630x; 1.5630x over previous
import functools

import jax
import jax.numpy as jnp
from jax import lax
from jax.experimental import pallas as pl
from jax.experimental.pallas import tpu as pltpu
from jax.experimental.pallas import tpu_sc as plsc

GRID_N = 16
NV = GRID_N * GRID_N
D = 768
B = 65536
NC, NS = 2, 16
NW = NC * NS
PER_W = B // NW
NBUF = 4
CH3 = 32
NCH3 = PER_W // CH3
BCH = 40
DCH = NCH3 - BCH
DLAG = 4

_MESH = plsc.VectorSubcoreMesh(core_axis_name="c", subcore_axis_name="s")


@functools.partial(
    pl.kernel,
    mesh=_MESH,
    out_type=jax.ShapeDtypeStruct((B, D), jnp.float32),
    scratch_types=[
        pltpu.VMEM_SHARED((NV * D,), jnp.float32),
        pltpu.VMEM((PER_W,), jnp.int32),
        pltpu.VMEM((NBUF, CH3, D), jnp.float32),
        pltpu.VMEM((GRID_N,), jnp.int32),
        pltpu.VMEM((GRID_N,), jnp.int32),
        pltpu.SemaphoreType.DMA,
        pltpu.SemaphoreType.DMA,
        pltpu.SemaphoreType.DMA,
        pltpu.SemaphoreType.DMA,
        pltpu.SemaphoreType.DMA,
        pltpu.SemaphoreType.DMA,
        pltpu.SemaphoreType.DMA,
        pltpu.SemaphoreType.DMA,
        pltpu.SemaphoreType.DMA,
    ],
)
def _embed(pos_hbm, crd0_hbm, crd1_hbm, emb0_hbm, emb1_hbm, out_hbm,
           table_sh, idx_v, bufs, crd0_v, crd1_v,
           sw0, sw1, sw2, sw3, sf0, sf1, sf2, sf3, sdr):
    cid = lax.axis_index("c")
    sid = lax.axis_index("s")
    wid = sid * NC + cid
    row0 = wid * PER_W
    sw = (sw0, sw1, sw2, sw3)
    sf = (sf0, sf1, sf2, sf3)

    t0 = sid * GRID_N
    pltpu.sync_copy(crd0_hbm.at[pl.ds(t0, GRID_N)], crd0_v)
    pltpu.sync_copy(crd1_hbm.at[pl.ds(t0, GRID_N)], crd1_v)
    c0 = crd0_v[...]
    c1 = crd1_v[...]
    cp0 = pltpu.async_copy(emb0_hbm.at[c0], bufs.at[0, pl.ds(0, GRID_N)], sf0)
    cp1 = pltpu.async_copy(emb1_hbm.at[c1], bufs.at[1, pl.ds(0, GRID_N)], sf1)
    pltpu.sync_copy(pos_hbm.at[pl.ds(row0, PER_W)], idx_v)
    cp0.wait()
    cp1.wait()

    def addrow(r, carry):
        for f in range(D // 16):
            sl = pl.ds(f * 16, 16)
            bufs[0, r, sl] = bufs[0, r, sl] + bufs[1, r, sl]
        return carry

    lax.fori_loop(0, GRID_N, addrow, 0)
    def stage(r, carry):
        pltpu.sync_copy(bufs.at[0, r], table_sh.at[pl.ds((t0 + r) * D, D)])
        return carry

    lax.fori_loop(0, GRID_N, stage, 0)
    plsc.subcore_barrier()

    def wstart(g, b):
        pltpu.async_copy(bufs.at[b],
                         out_hbm.at[pl.ds(row0 + g * CH3, CH3)], sw[b])

    def wwait(b):
        pltpu.make_async_copy(bufs.at[b],
                              out_hbm.at[pl.ds(row0, CH3)], sw[b]).wait()

    def fill_issue(cbase, b):
        def qbody(q, carry):
            pvec = idx_v[pl.ds(cbase + q * 16, 16)]
            for j in range(16):
                p = pvec[j] * D
                pltpu.async_copy(table_sh.at[pl.ds(p, D)],
                                 bufs.at[b, q * 16 + j], sf[b])
            return carry
        lax.fori_loop(0, CH3 // 16, qbody, 0)

    def fill_drain(b):
        pltpu.make_async_copy(out_hbm.at[pl.ds(row0, CH3)],
                              bufs.at[b], sf[b]).wait()

    def dissue(d, carry):
        cbase = (BCH + d) * CH3
        def qbody(q, carry2):
            pvec = idx_v[pl.ds(cbase + q * 16, 16)]
            for j in range(16):
                p = pvec[j] * D
                pltpu.async_copy(table_sh.at[pl.ds(p, D)],
                                 out_hbm.at[row0 + cbase + q * 16 + j], sdr)
            return carry2
        lax.fori_loop(0, CH3 // 16, qbody, 0)
        return carry

    def ddrain(carry):
        pltpu.make_async_copy(pos_hbm.at[pl.ds(0, CH3 * D)],
                              table_sh.at[pl.ds(0, CH3 * D)], sdr).wait()
        return carry

    fill_issue(0, 0)

    def body(k, carry):
        for par in range(NBUF):
            g = NBUF * k + par

            @pl.when(g + 1 < BCH)
            def _():
                nb = (par + 1) % NBUF

                @pl.when(g + 1 >= NBUF)
                def _():
                    wwait(nb)

                fill_issue((g + 1) * CH3, nb)

            fill_drain(par)
            wstart(g, par)

        dissue(2 * k, 0)
        dissue(2 * k + 1, 0)

        @pl.when(k >= DLAG // 2)
        def _():
            ddrain(0)
            ddrain(0)
        return carry

    lax.fori_loop(0, BCH // NBUF, body, 0)
    for i in range(DCH - 2 * (BCH // NBUF)):
        dissue(2 * (BCH // NBUF) + i, 0)
    lax.fori_loop(0, DLAG + DCH - 2 * (BCH // NBUF),
                  lambda k, c: ddrain(c), 0)

    for b in range(NBUF):
        wwait(b)


def kernel(positions, coords, emb0, emb1):
    pos = positions.astype(jnp.int32)
    crd = coords.astype(jnp.int32)
    return _embed(pos, crd[:, 0].reshape(-1), crd[:, 1].reshape(-1),
                  emb0.astype(jnp.float32), emb1.astype(jnp.float32))


# speedup vs baseline: 1.6466x; 1.0535x over previous
import functools

import jax
import jax.numpy as jnp
from jax import lax
from jax.experimental import pallas as pl
from jax.experimental.pallas import tpu as pltpu
from jax.experimental.pallas import tpu_sc as plsc

GRID_N = 16
NV = GRID_N * GRID_N
D = 768
B = 65536
NC, NS = 2, 16
NW = NC * NS
PER_W = B // NW
NBUF = 3
CH3 = 32
NCH3 = PER_W // CH3
BCH = 48
DCH = NCH3 - BCH
DLAG = 4

_MESH = plsc.VectorSubcoreMesh(core_axis_name="c", subcore_axis_name="s")


@functools.partial(
    pl.kernel,
    mesh=_MESH,
    out_type=jax.ShapeDtypeStruct((B, D), jnp.float32),
    scratch_types=[
        pltpu.VMEM_SHARED((NV * D,), jnp.float32),
        pltpu.VMEM((PER_W,), jnp.int32),
        pltpu.VMEM((NBUF, CH3, D), jnp.float32),
        pltpu.VMEM((GRID_N,), jnp.int32),
        pltpu.VMEM((GRID_N,), jnp.int32),
        pltpu.SemaphoreType.DMA,
        pltpu.SemaphoreType.DMA,
        pltpu.SemaphoreType.DMA,
        pltpu.SemaphoreType.DMA,
        pltpu.SemaphoreType.DMA,
        pltpu.SemaphoreType.DMA,
        pltpu.SemaphoreType.DMA,
    ],
)
def _embed(pos_hbm, crd0_hbm, crd1_hbm, emb0_hbm, emb1_hbm, out_hbm,
           table_sh, idx_v, bufs, crd0_v, crd1_v,
           sw0, sw1, sw2, sf0, sf1, sf2, sdr):
    cid = lax.axis_index("c")
    sid = lax.axis_index("s")
    wid = sid * NC + cid
    row0 = wid * PER_W
    sw = (sw0, sw1, sw2)
    sf = (sf0, sf1, sf2)

    t0 = sid * GRID_N
    pltpu.sync_copy(crd0_hbm.at[pl.ds(t0, GRID_N)], crd0_v)
    pltpu.sync_copy(crd1_hbm.at[pl.ds(t0, GRID_N)], crd1_v)
    c0 = crd0_v[...]
    c1 = crd1_v[...]
    cp0 = pltpu.async_copy(emb0_hbm.at[c0], bufs.at[0, pl.ds(0, GRID_N)], sf0)
    cp1 = pltpu.async_copy(emb1_hbm.at[c1], bufs.at[1, pl.ds(0, GRID_N)], sf1)
    pltpu.sync_copy(pos_hbm.at[pl.ds(row0, PER_W)], idx_v)
    cp0.wait()
    cp1.wait()

    def addrow(r, carry):
        for f in range(D // 16):
            sl = pl.ds(f * 16, 16)
            bufs[0, r, sl] = bufs[0, r, sl] + bufs[1, r, sl]
        return carry

    lax.fori_loop(0, GRID_N, addrow, 0)
    def stage(r, carry):
        pltpu.sync_copy(bufs.at[0, r], table_sh.at[pl.ds((t0 + r) * D, D)])
        return carry

    lax.fori_loop(0, GRID_N, stage, 0)
    plsc.subcore_barrier()

    def wstart(g, b):
        pltpu.async_copy(bufs.at[b],
                         out_hbm.at[pl.ds(row0 + g * CH3, CH3)], sw[b])

    def wwait(b):
        pltpu.make_async_copy(bufs.at[b],
                              out_hbm.at[pl.ds(row0, CH3)], sw[b]).wait()

    def fill_issue(cbase, b):
        def qbody(q, carry):
            pvec = idx_v[pl.ds(cbase + q * 16, 16)]
            for j in range(16):
                p = pvec[j] * D
                pltpu.async_copy(table_sh.at[pl.ds(p, D)],
                                 bufs.at[b, q * 16 + j], sf[b])
            return carry
        lax.fori_loop(0, CH3 // 16, qbody, 0)

    def fill_drain(b):
        pltpu.make_async_copy(out_hbm.at[pl.ds(row0, CH3)],
                              bufs.at[b], sf[b]).wait()

    def dissue(d, carry):
        cbase = (BCH + d) * CH3
        def qbody(q, carry2):
            pvec = idx_v[pl.ds(cbase + q * 16, 16)]
            for j in range(16):
                p = pvec[j] * D
                pltpu.async_copy(table_sh.at[pl.ds(p, D)],
                                 out_hbm.at[row0 + cbase + q * 16 + j], sdr)
            return carry2
        lax.fori_loop(0, CH3 // 16, qbody, 0)
        return carry

    def ddrain(carry):
        pltpu.make_async_copy(pos_hbm.at[pl.ds(0, CH3 * D)],
                              table_sh.at[pl.ds(0, CH3 * D)], sdr).wait()
        return carry

    fill_issue(0, 0)

    def body(k, carry):
        for par in range(NBUF):
            g = NBUF * k + par

            @pl.when(g + 1 < BCH)
            def _():
                nb = (par + 1) % NBUF

                @pl.when(g + 1 >= NBUF)
                def _():
                    wwait(nb)

                fill_issue((g + 1) * CH3, nb)

            fill_drain(par)
            wstart(g, par)

        dissue(k, 0)

        @pl.when(k >= DLAG)
        def _():
            ddrain(0)
        return carry

    lax.fori_loop(0, BCH // NBUF, body, 0)
    lax.fori_loop(0, DLAG, lambda k, c: ddrain(c), 0)

    for b in range(NBUF):
        wwait(b)


def kernel(positions, coords, emb0, emb1):
    pos = positions.astype(jnp.int32)
    crd = coords.astype(jnp.int32)
    return _embed(pos, crd[:, 0].reshape(-1), crd[:, 1].reshape(-1),
                  emb0.astype(jnp.float32), emb1.astype(jnp.float32))
